# baseline (speedup 1.0000x reference)
import functools

import jax
import jax.numpy as jnp
from jax import lax
from jax.experimental import pallas as pl
from jax.experimental.pallas import tpu as pltpu
from jax.experimental.pallas import tpu_sc as plsc

B = 4
S = 4096
HIDDEN = 2048
H4 = HIDDEN // 4
K = S // 2
TILE = 1024
NTILES = (B * S) // TILE
NV = S // 16


def _mlp_body(x_ref, w1_ref, b1_ref, w2_ref, b2_ref, scores_ref):
    x = x_ref[...]
    h = jnp.dot(x, w1_ref[...], preferred_element_type=jnp.float32) + b1_ref[...]
    g = h * (0.5 * (jax.lax.erf(h * jnp.float32(0.7071067811865476)) + 1.0))
    logits = jnp.dot(g, w2_ref[...], preferred_element_type=jnp.float32) + b2_ref[...]
    scores_ref[...] = jax.nn.sigmoid(logits)


def _lane_total(acc):
    c = acc[0]
    for t in range(1, 16):
        c = c + acc[t]
    return c


def _sc_select_body(scores_hbm, w_hbm, m_hbm, srow, wrow, mrow):
    cid = lax.axis_index("c")
    sid = lax.axis_index("s")
    wid = sid * 2 + cid

    @pl.when(wid < B)
    def _():
        row = wid
        pltpu.sync_copy(scores_hbm.at[pl.ds(row * S, S)], srow)

        def keyvec(j):
            return lax.bitcast_convert_type(srow[pl.ds(j * 16, 16)], jnp.int32)

        def count_ge_shifted(shift, qtop):
            def body(j, acc):
                m = (keyvec(j) >> shift) >= qtop
                return acc + jnp.where(m, 1, 0)

            acc = lax.fori_loop(0, NV, body, jnp.zeros((16,), jnp.int32),
                                unroll=8)
            return _lane_total(acc)

        def step(it, p):
            b = 30 - it
            q = p | (jnp.int32(1) << b)
            c = count_ge_shifted(b, q >> b)
            return jnp.where(c >= K, q, p)

        p = lax.fori_loop(0, 31, step, jnp.int32(0))

        def gt_eq_body(j, acc):
            kv = keyvec(j)
            return (acc[0] + jnp.where(kv > p, 1, 0),
                    acc[1] + jnp.where(kv == p, 1, 0))

        acc0 = (jnp.zeros((16,), jnp.int32), jnp.zeros((16,), jnp.int32))
        accg, acce = lax.fori_loop(0, NV, gt_eq_body, acc0, unroll=8)
        n_gt = _lane_total(accg)
        n_eq = _lane_total(acce)
        need = K - n_gt

        def tie_descent():
            def step2(it, plo):
                b = 11 - it
                q = plo | (jnp.int32(1) << b)

                def body(j, acc):
                    kv = keyvec(j)
                    lov = (S - 1 - j * 16) - lax.iota(jnp.int32, 16)
                    m = (kv == p) & ((lov >> b) >= (q >> b))
                    return acc + jnp.where(m, 1, 0)

                c = _lane_total(lax.fori_loop(
                    0, NV, body, jnp.zeros((16,), jnp.int32), unroll=8))
                return jnp.where(c >= need, q, plo)

            return lax.fori_loop(0, 12, step2, jnp.int32(0))

        plo = lax.cond(n_eq == need, lambda: jnp.int32(0), tie_descent)

        def out_body(j, carry):
            kv = keyvec(j)
            lov = (S - 1 - j * 16) - lax.iota(jnp.int32, 16)
            m = (kv > p) | ((kv == p) & (lov >= plo))
            sl = pl.ds(j * 16, 16)
            mrow[sl] = jnp.where(m, 1, 0)
            wrow[sl] = jnp.where(m, srow[sl], 0.0)
            return carry

        lax.fori_loop(0, NV, out_body, jnp.int32(0), unroll=8)

        pltpu.sync_copy(wrow, w_hbm.at[pl.ds(row * S, S)])
        pltpu.sync_copy(mrow, m_hbm.at[pl.ds(row * S, S)])


_sc_select = functools.partial(
    pl.kernel,
    out_type=(
        jax.ShapeDtypeStruct((B * S,), jnp.float32),
        jax.ShapeDtypeStruct((B * S,), jnp.int32),
    ),
    mesh=plsc.VectorSubcoreMesh(core_axis_name="c", subcore_axis_name="s"),
    scratch_types=[
        pltpu.VMEM((S,), jnp.float32),
        pltpu.VMEM((S,), jnp.float32),
        pltpu.VMEM((S,), jnp.int32),
    ],
)(_sc_select_body)


@jax.jit
def kernel(hidden_states, W1, b1, W2, b2):
    x = hidden_states.reshape(B * S, HIDDEN)
    scores = pl.pallas_call(
        _mlp_body,
        grid=(NTILES,),
        in_specs=[
            pl.BlockSpec((TILE, HIDDEN), lambda i: (i, 0)),
            pl.BlockSpec((HIDDEN, H4), lambda i: (0, 0)),
            pl.BlockSpec((1, H4), lambda i: (0, 0)),
            pl.BlockSpec((H4, 1), lambda i: (0, 0)),
            pl.BlockSpec((1, 1), lambda i: (0, 0)),
        ],
        out_specs=pl.BlockSpec((TILE, 1), lambda i: (i, 0)),
        out_shape=jax.ShapeDtypeStruct((B * S, 1), jnp.float32),
        compiler_params=pltpu.CompilerParams(
            dimension_semantics=("arbitrary",)),
    )(x, W1, b1.reshape(1, H4), W2, b2.reshape(1, 1))

    weights, mask = _sc_select(scores.reshape(B * S))
    return weights.reshape(B, S), mask.reshape(B, S).astype(jnp.bool_)


# speedup vs baseline: 1.5283x; 1.5283x over previous
import jax
import jax.numpy as jnp
from jax.experimental import pallas as pl
from jax.experimental.pallas import tpu as pltpu

B = 4
S = 4096
HIDDEN = 2048
H4 = HIDDEN // 4
K = S // 2
TILE = 1024
NTILES = (B * S) // TILE
RPT = S // TILE


def _body(x_ref, w1_ref, b1_ref, w2t_ref, b2_ref, w_ref, m_ref, scores_ref):
    i = pl.program_id(0)
    x = x_ref[...]
    h = jnp.dot(x, w1_ref[...], preferred_element_type=jnp.float32) + b1_ref[...]
    g = h * (0.5 * (jax.lax.erf(h * jnp.float32(0.7071067811865476)) + 1.0))
    logits = jax.lax.dot_general(
        w2t_ref[...], g, (((1,), (1,)), ((), ())),
        preferred_element_type=jnp.float32) + b2_ref[...]
    row = i // RPT
    col = pl.multiple_of((i % RPT) * TILE, TILE)
    scores_ref[pl.ds(row, 1), pl.ds(col, TILE)] = jax.nn.sigmoid(logits)

    @pl.when(i == NTILES - 1)
    def _select():
        s = scores_ref[...]
        key = jax.lax.bitcast_convert_type(s, jnp.int32)

        def step(it, p):
            b = 30 - it
            q = p | (1 << b)
            c = jnp.sum(((key >> b) >= (q >> b)).astype(jnp.int32), axis=1,
                        keepdims=True)
            return jnp.where(c >= K, q, p)

        p = jax.lax.fori_loop(0, 31, step, jnp.zeros((B, 1), jnp.int32))

        gt = key > p
        eq = key == p
        need = K - jnp.sum(gt.astype(jnp.int32), axis=1, keepdims=True)

        lo = (S - 1) - jax.lax.broadcasted_iota(jnp.int32, (B, S), 1)

        def step2(it, plo):
            b = 11 - it
            q = plo | (1 << b)
            c = jnp.sum((eq & ((lo >> b) >= (q >> b))).astype(jnp.int32),
                        axis=1, keepdims=True)
            return jnp.where(c >= need, q, plo)

        plo = jax.lax.fori_loop(0, 12, step2, jnp.zeros((B, 1), jnp.int32))

        mask = gt | (eq & (lo >= plo))
        m_ref[...] = mask
        w_ref[...] = s * mask.astype(s.dtype)


@jax.jit
def kernel(hidden_states, W1, b1, W2, b2):
    x = hidden_states.reshape(B * S, HIDDEN)
    weights, mask = pl.pallas_call(
        _body,
        grid=(NTILES,),
        in_specs=[
            pl.BlockSpec((TILE, HIDDEN), lambda i: (i, 0)),
            pl.BlockSpec((HIDDEN, H4), lambda i: (0, 0)),
            pl.BlockSpec((1, H4), lambda i: (0, 0)),
            pl.BlockSpec((1, H4), lambda i: (0, 0)),
            pl.BlockSpec((1, 1), lambda i: (0, 0)),
        ],
        out_specs=(
            pl.BlockSpec((B, S), lambda i: (0, 0)),
            pl.BlockSpec((B, S), lambda i: (0, 0)),
        ),
        out_shape=(
            jax.ShapeDtypeStruct((B, S), jnp.float32),
            jax.ShapeDtypeStruct((B, S), jnp.bool_),
        ),
        scratch_shapes=[pltpu.VMEM((B, S), jnp.float32)],
        compiler_params=pltpu.CompilerParams(
            dimension_semantics=("arbitrary",)),
    )(x, W1, b1.reshape(1, H4), W2.reshape(1, H4), b2.reshape(1, 1))
    return weights, mask
